# baseline (speedup 1.0000x reference)
import jax
import jax.numpy as jnp
from jax.experimental import pallas as pl
from jax.experimental.pallas import tpu as pltpu

F32 = jnp.float32
BF16 = jnp.bfloat16


def _dot(a, b):
    return jax.lax.dot_general(a, b, (((1,), (0,)), ((), ())),
                               preferred_element_type=F32)


def _support_body(x_ref, w_ref, out_ref):
    out_ref[:] = _dot(x_ref[:], w_ref[:]).astype(BF16)


def _layer1_body(adj_ref, s_ref, b_ref, w_ref, adjb_ref, snext_ref):
    a = adj_ref[:].astype(BF16)
    adjb_ref[:] = a
    h = jnp.maximum(_dot(a, s_ref[:]) + b_ref[:], 0.0)
    snext_ref[:] = _dot(h, w_ref[:]).astype(BF16)


def _layer_mid_body(adj_ref, s_ref, b_ref, w_ref, snext_ref):
    h = jnp.maximum(_dot(adj_ref[:], s_ref[:]) + b_ref[:], 0.0)
    snext_ref[:] = _dot(h, w_ref[:]).astype(BF16)


def _layer4_body(adj_ref, s_ref, b_ref, emb_ref, embb_ref):
    h = jnp.maximum(_dot(adj_ref[:], s_ref[:]) + b_ref[:], 0.0)
    emb_ref[:] = h
    embb_ref[:] = h.astype(BF16)


def _decoder_body(emb_ref, out_ref):
    i = pl.program_id(0)
    bm = out_ref.shape[0]
    blk = emb_ref[pl.ds(i * bm, bm), :]
    logits = jax.lax.dot_general(blk, emb_ref[:], (((1,), (1,)), ((), ())),
                                 preferred_element_type=F32)
    out_ref[:] = jax.nn.sigmoid(logits)


def _params(n_par):
    return pltpu.CompilerParams(dimension_semantics=("parallel",) * n_par)


def kernel(x, adj, W1, b1, W2, b2, W3, b3, W4, b4):
    N, D = x.shape
    H1, H2 = W2.shape
    H3, Z = W4.shape
    b1r, b2r = b1.reshape(1, H1), b2.reshape(1, H2)
    b3r, b4r = b3.reshape(1, H3), b4.reshape(1, Z)

    bs = min(512, N)
    s1 = pl.pallas_call(
        _support_body,
        grid=(N // bs,),
        in_specs=[pl.BlockSpec((bs, D), lambda i: (i, 0)),
                  pl.BlockSpec((D, H1), lambda i: (0, 0))],
        out_specs=pl.BlockSpec((bs, H1), lambda i: (i, 0)),
        out_shape=jax.ShapeDtypeStruct((N, H1), BF16),
        compiler_params=_params(1),
    )(x, W1)

    bm = min(256, N)
    nb = N // bm
    adj16, s2 = pl.pallas_call(
        _layer1_body,
        grid=(nb,),
        in_specs=[pl.BlockSpec((bm, N), lambda i: (i, 0)),
                  pl.BlockSpec((N, H1), lambda i: (0, 0)),
                  pl.BlockSpec((1, H1), lambda i: (0, 0)),
                  pl.BlockSpec((H1, H2), lambda i: (0, 0))],
        out_specs=[pl.BlockSpec((bm, N), lambda i: (i, 0)),
                   pl.BlockSpec((bm, H2), lambda i: (i, 0))],
        out_shape=[jax.ShapeDtypeStruct((N, N), BF16),
                   jax.ShapeDtypeStruct((N, H2), BF16)],
        compiler_params=_params(1),
    )(adj, s1, b1r, W2)

    def mid_layer(s, br, Wn, h_in, h_out):
        return pl.pallas_call(
            _layer_mid_body,
            grid=(nb,),
            in_specs=[pl.BlockSpec((bm, N), lambda i: (i, 0)),
                      pl.BlockSpec((N, h_in), lambda i: (0, 0)),
                      pl.BlockSpec((1, h_in), lambda i: (0, 0)),
                      pl.BlockSpec((h_in, h_out), lambda i: (0, 0))],
            out_specs=pl.BlockSpec((bm, h_out), lambda i: (i, 0)),
            out_shape=jax.ShapeDtypeStruct((N, h_out), BF16),
            compiler_params=_params(1),
        )(adj16, s, br, Wn)

    s3 = mid_layer(s2, b2r, W3, H2, H3)
    s4 = mid_layer(s3, b3r, W4, H3, Z)

    emb, emb16 = pl.pallas_call(
        _layer4_body,
        grid=(nb,),
        in_specs=[pl.BlockSpec((bm, N), lambda i: (i, 0)),
                  pl.BlockSpec((N, Z), lambda i: (0, 0)),
                  pl.BlockSpec((1, Z), lambda i: (0, 0))],
        out_specs=[pl.BlockSpec((bm, Z), lambda i: (i, 0)),
                   pl.BlockSpec((bm, Z), lambda i: (i, 0))],
        out_shape=[jax.ShapeDtypeStruct((N, Z), F32),
                   jax.ShapeDtypeStruct((N, Z), BF16)],
        compiler_params=_params(1),
    )(adj16, s4, b4r)

    bd = min(256, N)
    adj_hat = pl.pallas_call(
        _decoder_body,
        grid=(N // bd,),
        in_specs=[pl.BlockSpec((N, Z), lambda i: (0, 0))],
        out_specs=pl.BlockSpec((bd, N), lambda i: (i, 0)),
        out_shape=jax.ShapeDtypeStruct((N, N), F32),
        compiler_params=_params(1),
    )(emb16)

    return (emb, adj_hat)
